# Initial kernel scaffold; baseline (speedup 1.0000x reference)
import functools

import jax
import jax.numpy as jnp
from jax import lax
from jax.experimental import pallas as pl
from jax.experimental.pallas import tpu as pltpu
from jax.experimental.pallas import tpu_sc as plsc

NC = 2
NS = 16
NW = NC * NS

G = 128
K = 8
BLK = K * G


def _gather_sc(table, idx2d, n, d):
    nb = n // NW
    steps = nb // BLK

    mesh = plsc.VectorSubcoreMesh(core_axis_name="c", subcore_axis_name="s")

    @functools.partial(
        pl.kernel,
        out_type=jax.ShapeDtypeStruct((n, d), jnp.float32),
        mesh=mesh,
        scratch_types=[
            pltpu.VMEM((K, G), jnp.int32),
            pltpu.VMEM((BLK, d), jnp.float32),
            pltpu.SemaphoreType.DMA,
        ],
    )
    def k(table_hbm, idx_hbm, out_hbm, idx_v, rows_v, sem):
        wid = lax.axis_index("s") * NC + lax.axis_index("c")
        base = wid * steps

        def blk(i, carry):
            b = base + i
            pltpu.sync_copy(idx_hbm.at[pl.ds(b * K, K)], idx_v)
            handles = []
            for j in range(K):
                handles.append(
                    pltpu.async_copy(
                        table_hbm.at[idx_v.at[j]],
                        rows_v.at[pl.ds(j * G, G)],
                        sem,
                    )
                )
            for h in handles:
                h.wait()
            pltpu.sync_copy(rows_v, out_hbm.at[pl.ds(b * BLK, BLK)])
            return carry

        lax.fori_loop(0, steps, blk, 0)

    return k(table, idx2d)


def kernel(message, weight):
    b, h = message.shape
    v, d = weight.shape
    n = b * h
    idx2d = message.reshape(n // G, G)
    out = _gather_sc(weight, idx2d, n, d)
    return out.reshape(b, h, d)


# speedup vs baseline: 4.8108x; 4.8108x over previous
import functools

import jax
import jax.numpy as jnp
from jax import lax
from jax.experimental import pallas as pl
from jax.experimental.pallas import tpu as pltpu
from jax.experimental.pallas import tpu_sc as plsc

NC = 2
NS = 16
NW = NC * NS

G = 128
K = 8
BLK = K * G


def _gather_sc(table, idx2d, n, d):
    nb = n // NW
    steps = nb // BLK

    mesh = plsc.VectorSubcoreMesh(core_axis_name="c", subcore_axis_name="s")

    @functools.partial(
        pl.kernel,
        out_type=jax.ShapeDtypeStruct((n, d), jnp.float32),
        mesh=mesh,
        scratch_types=[
            pltpu.VMEM((K, G), jnp.int32),
            pltpu.VMEM((BLK, d), jnp.float32),
            pltpu.SemaphoreType.DMA,
        ],
        compiler_params=pltpu.CompilerParams(use_tc_tiling_on_sc=False),
    )
    def k(table_hbm, idx_hbm, out_hbm, idx_v, rows_v, sem):
        wid = lax.axis_index("s") * NC + lax.axis_index("c")
        base = wid * steps

        def blk(i, carry):
            b = base + i
            pltpu.sync_copy(idx_hbm.at[pl.ds(b * K, K)], idx_v)
            handles = []
            for j in range(K):
                handles.append(
                    pltpu.async_copy(
                        table_hbm.at[idx_v.at[j]],
                        rows_v.at[pl.ds(j * G, G)],
                        sem,
                    )
                )
            for h in handles:
                h.wait()
            pltpu.sync_copy(rows_v, out_hbm.at[pl.ds(b * BLK, BLK)])
            return carry

        lax.fori_loop(0, steps, blk, 0)

    return k(table, idx2d)


def kernel(message, weight):
    b, h = message.shape
    v, d = weight.shape
    n = b * h
    idx2d = message.reshape(n // G, G)
    out = _gather_sc(weight, idx2d, n, d)
    return out.reshape(b, h, d)


# speedup vs baseline: 5.0328x; 1.0461x over previous
import functools

import jax
import jax.numpy as jnp
from jax import lax
from jax.experimental import pallas as pl
from jax.experimental.pallas import tpu as pltpu
from jax.experimental.pallas import tpu_sc as plsc

NC = 2
NS = 16
NW = NC * NS

G = 128
K = 8
BLK = K * G
UNROLL = 4


def _gather_sc(table, idx2d, n, d):
    nb = n // NW
    steps = nb // BLK
    ni = steps // UNROLL

    mesh = plsc.VectorSubcoreMesh(core_axis_name="c", subcore_axis_name="s")

    @functools.partial(
        pl.kernel,
        out_type=jax.ShapeDtypeStruct((n, d), jnp.float32),
        mesh=mesh,
        scratch_types=[
            pltpu.VMEM((K, G), jnp.int32),
            pltpu.VMEM((K, G), jnp.int32),
            pltpu.VMEM((K, G), jnp.int32),
            pltpu.VMEM((K, G), jnp.int32),
            pltpu.VMEM((BLK, d), jnp.float32),
            pltpu.VMEM((BLK, d), jnp.float32),
            pltpu.SemaphoreType.DMA,
            pltpu.SemaphoreType.DMA,
            pltpu.SemaphoreType.DMA,
        ],
        compiler_params=pltpu.CompilerParams(use_tc_tiling_on_sc=False),
    )
    def k(table_hbm, idx_hbm, out_hbm, i0, i1, i2, i3, r0, r1,
          sem_i, sem_g, sem_o):
        ibufs = (i0, i1, i2, i3)
        rbufs = (r0, r1)
        wid = lax.axis_index("s") * NC + lax.axis_index("c")
        base = wid * steps

        pltpu.async_copy(idx_hbm.at[pl.ds(base * K, K)], i0, sem_i)
        pltpu.async_copy(idx_hbm.at[pl.ds((base + 1) * K, K)], i1, sem_i)

        def body(i, carry):
            for u in range(UNROLL):
                gb = base + i * UNROLL + u
                ib = ibufs[u]
                rb = rbufs[u % 2]

                pltpu.make_async_copy(
                    idx_hbm.at[pl.ds(gb * K, K)], ib, sem_i).wait()

                def drain_out():
                    pltpu.make_async_copy(
                        rb, out_hbm.at[pl.ds((gb - 2) * BLK, BLK)],
                        sem_o).wait()
                if u >= 2:
                    drain_out()
                else:
                    pl.when(i > 0)(drain_out)

                hs = [
                    pltpu.async_copy(
                        table_hbm.at[ib.at[j]],
                        rb.at[pl.ds(j * G, G)],
                        sem_g,
                    )
                    for j in range(K)
                ]

                def prefetch():
                    pltpu.async_copy(
                        idx_hbm.at[pl.ds((gb + 2) * K, K)],
                        ibufs[(u + 2) % 4], sem_i)
                if u < 2:
                    prefetch()
                else:
                    pl.when(i < ni - 1)(prefetch)

                for h in hs:
                    h.wait()

                pltpu.async_copy(rb, out_hbm.at[pl.ds(gb * BLK, BLK)], sem_o)
            return carry

        lax.fori_loop(0, ni, body, 0)

        last = base + steps
        pltpu.make_async_copy(
            r0, out_hbm.at[pl.ds((last - 2) * BLK, BLK)], sem_o).wait()
        pltpu.make_async_copy(
            r1, out_hbm.at[pl.ds((last - 1) * BLK, BLK)], sem_o).wait()

    return k(table, idx2d)


def kernel(message, weight):
    b, h = message.shape
    v, d = weight.shape
    n = b * h
    idx2d = message.reshape(n // G, G)
    out = _gather_sc(weight, idx2d, n, d)
    return out.reshape(b, h, d)
